# baseline (speedup 1.0000x reference)
import functools

import jax
import jax.numpy as jnp
from jax import lax
from jax.experimental import pallas as pl
from jax.experimental.pallas import tpu as pltpu
from jax.experimental.pallas import tpu_sc as plsc

HIDDEN = 64
BATCH = 16384
NUM_WORKERS = 32
B_PER_W = BATCH // NUM_WORKERS
CHUNK = 128
NCHUNK = B_PER_W // CHUNK


def _sc_gather2(embedding, old_token, new_token):
  mesh = plsc.VectorSubcoreMesh(core_axis_name="c", subcore_axis_name="s")

  @functools.partial(
      pl.kernel,
      out_type=[jax.ShapeDtypeStruct((BATCH, HIDDEN), jnp.float32),
                jax.ShapeDtypeStruct((BATCH, HIDDEN), jnp.float32)],
      mesh=mesh,
      scratch_types=[
          pltpu.VMEM((B_PER_W,), jnp.int32),
          pltpu.VMEM((B_PER_W,), jnp.int32),
          pltpu.VMEM((B_PER_W, HIDDEN), jnp.float32),
          pltpu.VMEM((B_PER_W, HIDDEN), jnp.float32),
          pltpu.SemaphoreType.DMA,
      ],
      compiler_params=pltpu.CompilerParams(use_tc_tiling_on_sc=False),
  )
  def k(table, old_hbm, new_hbm, x_hbm, y_hbm, oidx, nidx, xrows, yrows, sem):
    wid = lax.axis_index("s") * 2 + lax.axis_index("c")
    base = wid * B_PER_W
    pltpu.sync_copy(old_hbm.at[pl.ds(base, B_PER_W)], oidx)
    pltpu.sync_copy(new_hbm.at[pl.ds(base, B_PER_W)], nidx)
    copies = []
    for j in range(NCHUNK):
      sl = pl.ds(j * CHUNK, CHUNK)
      copies.append(pltpu.async_copy(table.at[oidx.at[sl]], xrows.at[sl], sem))
      copies.append(pltpu.async_copy(table.at[nidx.at[sl]], yrows.at[sl], sem))
    for c in copies:
      c.wait()
    pltpu.sync_copy(xrows, x_hbm.at[pl.ds(base, B_PER_W)])
    pltpu.sync_copy(yrows, y_hbm.at[pl.ds(base, B_PER_W)])

  return k(embedding, old_token, new_token)


def _combine_body(h_ref, x_ref, y_ref, pos_ref, gamma_ref, lin_ref, g_ref,
                  b_ref, o_ref):
  pf = pos_ref[...]
  scale = (g_ref[...] * jnp.power(gamma_ref[...], pf)
           + pf * lin_ref[...] + b_ref[...])
  o_ref[...] = h_ref[...] + (jnp.tanh(y_ref[...]) - jnp.tanh(x_ref[...])) * scale


def _tc_combine(h, x, y, pos_f, gamma, lin, g, b):
  BS = 2048
  row = pl.BlockSpec((BS, HIDDEN), lambda i: (i, 0))
  vec = pl.BlockSpec((1, HIDDEN), lambda i: (0, 0))
  return pl.pallas_call(
      _combine_body,
      grid=(BATCH // BS,),
      in_specs=[row, row, row, pl.BlockSpec((BS, 1), lambda i: (i, 0)),
                vec, vec, vec, vec],
      out_specs=row,
      out_shape=jax.ShapeDtypeStruct((BATCH, HIDDEN), jnp.float32),
  )(h, x, y, pos_f, gamma.reshape(1, HIDDEN), lin.reshape(1, HIDDEN),
    g.reshape(1, HIDDEN), b.reshape(1, HIDDEN))


def kernel(h, old_token, new_token, pos, embedding, gamma, lin, g, b):
  x, y = _sc_gather2(embedding, old_token, new_token)
  pos_f = pos.astype(jnp.float32).reshape(BATCH, 1)
  return _tc_combine(h, x, y, pos_f, gamma, lin, g, b)


# speedup vs baseline: 1.5972x; 1.5972x over previous
import functools

import jax
import jax.numpy as jnp
from jax import lax
from jax.experimental import pallas as pl
from jax.experimental.pallas import tpu as pltpu
from jax.experimental.pallas import tpu_sc as plsc

HIDDEN = 64
BATCH = 16384
NUM_WORKERS = 32
B_PER_W = BATCH // NUM_WORKERS
CHUNK = 32
NCHUNK = B_PER_W // CHUNK
TPC = CHUNK // 8


def _tanh(v):
  return 1.0 - 2.0 / (jnp.exp(2.0 * v) + 1.0)


def _sc_fused(emb3, old_token, new_token, pos, h3, lin, g, b, lg):
  mesh = plsc.VectorSubcoreMesh(core_axis_name="c", subcore_axis_name="s")

  @functools.partial(
      pl.kernel,
      out_type=jax.ShapeDtypeStruct((BATCH // 8, 8, HIDDEN), jnp.float32),
      mesh=mesh,
      scratch_types=[
          pltpu.VMEM((B_PER_W + 16,), jnp.int32),
          pltpu.VMEM((B_PER_W + 16,), jnp.int32),
          pltpu.VMEM((B_PER_W + 16,), jnp.int32),
          pltpu.VMEM((HIDDEN,), jnp.float32),
          pltpu.VMEM((HIDDEN,), jnp.float32),
          pltpu.VMEM((HIDDEN,), jnp.float32),
          pltpu.VMEM((HIDDEN,), jnp.float32),
          pltpu.VMEM((CHUNK, HIDDEN), jnp.float32),
          pltpu.VMEM((CHUNK, HIDDEN), jnp.float32),
          pltpu.VMEM((TPC, 8, HIDDEN), jnp.float32),
          pltpu.VMEM((TPC, 8, HIDDEN), jnp.float32),
          pltpu.SemaphoreType.DMA,
      ],
      compiler_params=pltpu.CompilerParams(use_tc_tiling_on_sc=True),
  )
  def k(table, old_hbm, new_hbm, pos_hbm, h_hbm, lin_hbm, g_hbm, b_hbm,
        lg_hbm, out_hbm, oidx, nidx, posv,
        gv, linv, bv, lgv, xt, yt, hb, ob, sem):
    wid = lax.axis_index("s") * 2 + lax.axis_index("c")
    base = wid * B_PER_W
    pltpu.sync_copy(old_hbm.at[pl.ds(base, B_PER_W)], oidx.at[pl.ds(0, B_PER_W)])
    pltpu.sync_copy(new_hbm.at[pl.ds(base, B_PER_W)], nidx.at[pl.ds(0, B_PER_W)])
    pltpu.sync_copy(pos_hbm.at[pl.ds(base, B_PER_W)], posv.at[pl.ds(0, B_PER_W)])
    pltpu.sync_copy(g_hbm, gv)
    pltpu.sync_copy(lin_hbm, linv)
    pltpu.sync_copy(b_hbm, bv)
    pltpu.sync_copy(lg_hbm, lgv)

    gvec = [gv[pl.ds(16 * j, 16)] for j in range(4)]
    linvec = [linv[pl.ds(16 * j, 16)] for j in range(4)]
    bvec = [bv[pl.ds(16 * j, 16)] for j in range(4)]
    lgvec = [lgv[pl.ds(16 * j, 16)] for j in range(4)]

    for c in range(NCHUNK):
      cb = c * CHUNK

      def issue(i, _):
        gi = cb + i
        orow = oidx[pl.ds(gi, 16)][0]
        nrow = nidx[pl.ds(gi, 16)][0]
        pltpu.async_copy(table.at[pl.ds(orow, 1), :], xt.at[pl.ds(i, 1), :],
                         sem)
        pltpu.async_copy(table.at[pl.ds(nrow, 1), :], yt.at[pl.ds(i, 1), :],
                         sem)
        return 0
      lax.fori_loop(0, CHUNK, issue, 0)
      pltpu.sync_copy(h_hbm.at[pl.ds(base // 8 + c * TPC, TPC)], hb)
      pltpu.make_async_copy(table.at[pl.ds(0, CHUNK), :], xt, sem).wait()
      pltpu.make_async_copy(table.at[pl.ds(0, CHUNK), :], yt, sem).wait()

      def body(i, _):
        gi = cb + i
        pf = posv[pl.ds(gi, 16)][0].astype(jnp.float32)
        it = lax.shift_right_logical(i, 3)
        is_ = lax.bitwise_and(i, 7)
        for j in range(4):
          sl = pl.ds(16 * j, 16)
          xv = xt[i, sl]
          yv = yt[i, sl]
          hv = hb[it, is_, sl]
          scale = gvec[j] * jnp.exp(pf * lgvec[j]) + pf * linvec[j] + bvec[j]
          ob[it, is_, sl] = hv + (_tanh(yv) - _tanh(xv)) * scale
        return 0
      lax.fori_loop(0, CHUNK, body, 0)

      pltpu.sync_copy(ob, out_hbm.at[pl.ds(base // 8 + c * TPC, TPC)])

  return k(emb3, old_token, new_token, pos, h3, lin, g, b, lg)


def kernel(h, old_token, new_token, pos, embedding, gamma, lin, g, b):
  h3 = h.reshape(BATCH // 8, 8, HIDDEN)
  lg = jnp.log(gamma)
  out3 = _sc_fused(embedding, old_token, new_token, pos, h3, lin, g, b, lg)
  return out3.reshape(BATCH, HIDDEN)
